# speedup vs baseline: 11.5422x; 11.5422x over previous
import functools

import jax
import jax.numpy as jnp
from jax.experimental import pallas as pl
from jax.experimental.pallas import tpu as pltpu

R_C = 6.0
K_MAX = 8
N_TYPES = 4
N_DESC = 16
E = 800000

B = 6400
NB = E // B


def _body(r_ref, ti_ref, tj_ref, w_ref, o_ref):
    r = r_ref[0, 0]
    p = ti_ref[0, 0] * N_TYPES + tj_ref[0, 0]

    fc = jnp.where(r < R_C, 0.5 * jnp.cos(jnp.pi * r / R_C) + 0.5, 0.0)
    x = 2.0 * (r / R_C - 1.0) ** 2 - 1.0
    half_fc = 0.5 * fc

    col = jax.lax.broadcasted_iota(jnp.int32, (1, 128), 1)
    j = col - 8 * p[:, None]

    xb = x[:, None] * jnp.ones((1, 128), jnp.float32)
    hb = half_fc[:, None] * jnp.ones((1, 128), jnp.float32)

    two_x = 2.0 * xb
    cur = jnp.ones_like(xb)
    prev = jnp.zeros_like(xb)
    acc = jnp.zeros_like(xb)
    for k in range(K_MAX):
        acc = jnp.where(j == k, cur, acc)
        if k == 0:
            cur, prev = xb, cur
        else:
            cur, prev = two_x * cur - prev, cur

    valid = (j >= 0) & (j < K_MAX)
    f_ext = jnp.where(valid, (acc + 1.0) * hb, 0.0)

    g = jax.lax.dot_general(
        f_ext, w_ref[...],
        dimension_numbers=(((1,), (0,)), ((), ())),
        preferred_element_type=jnp.float32,
        precision=jax.lax.Precision.HIGHEST,
    )
    o_ref[0] = g


@jax.jit
def kernel(r_ij, type_i, type_j, c_table):
    w = c_table.reshape(16, N_DESC, K_MAX).transpose(0, 2, 1).reshape(128, N_DESC)
    r3 = r_ij.reshape(NB, 1, B)
    ti3 = type_i.reshape(NB, 1, B)
    tj3 = type_j.reshape(NB, 1, B)

    grid_spec = pl.GridSpec(
        grid=(NB,),
        in_specs=[
            pl.BlockSpec((1, 1, B), lambda i: (i, 0, 0)),
            pl.BlockSpec((1, 1, B), lambda i: (i, 0, 0)),
            pl.BlockSpec((1, 1, B), lambda i: (i, 0, 0)),
            pl.BlockSpec((128, N_DESC), lambda i: (0, 0)),
        ],
        out_specs=pl.BlockSpec((1, B, N_DESC), lambda i: (i, 0, 0)),
    )
    out = pl.pallas_call(
        _body,
        grid_spec=grid_spec,
        out_shape=jax.ShapeDtypeStruct((NB, B, N_DESC), jnp.float32),
    )(r3, ti3, tj3, w)
    return out.reshape(E, N_DESC)
